# Initial kernel scaffold; baseline (speedup 1.0000x reference)
import jax
import jax.numpy as jnp
from jax.experimental import pallas as pl


def kernel(x, structural_indices, W, b):
    raise NotImplementedError("write your pallas kernel here")


# speedup vs baseline: 3.0850x; 3.0850x over previous
import functools

import jax
import jax.numpy as jnp
from jax.experimental import pallas as pl

N_ATOMS = 131072
D_FEAT = 512
D_OUT = 64
N_STRUCT = 1024

BLOCK_ATOMS = 1024
SEG_WINDOW = 128


def _fused_kernel(ids_ref, x_ref, w_ref, b_ref, out_ref):
    i = pl.program_id(0)

    @pl.when(i == 0)
    def _init():
        out_ref[...] = jnp.zeros_like(out_ref)

    xb = x_ref[...]
    yb = jnp.dot(xb, w_ref[...], preferred_element_type=jnp.float32)
    yb = yb + b_ref[...]

    ids = ids_ref[0, 0, :]
    min_id = jnp.min(ids)
    max_id = jnp.max(ids)
    nwin = (max_id - min_id) // SEG_WINDOW + 1

    def body(k, carry):
        win_lo = min_id + k * SEG_WINDOW
        store_base = jnp.minimum(win_lo, N_STRUCT - SEG_WINDOW)
        rel = ids - store_base
        sel = (ids >= win_lo) & (ids < win_lo + SEG_WINDOW)
        rows = jax.lax.broadcasted_iota(jnp.int32, (SEG_WINDOW, BLOCK_ATOMS), 0)
        oh = (rows == rel[None, :]) & sel[None, :]
        part = jnp.dot(oh.astype(jnp.float32), yb,
                       preferred_element_type=jnp.float32)
        out_ref[pl.ds(store_base, SEG_WINDOW), :] += part
        return carry

    jax.lax.fori_loop(0, nwin, body, 0)


@jax.jit
def kernel(x, structural_indices, W, b):
    nb = N_ATOMS // BLOCK_ATOMS
    ids3 = structural_indices.astype(jnp.int32).reshape(nb, 1, BLOCK_ATOMS)
    b2 = b.reshape(1, D_OUT)
    out = pl.pallas_call(
        _fused_kernel,
        grid=(nb,),
        in_specs=[
            pl.BlockSpec((1, 1, BLOCK_ATOMS), lambda i: (i, 0, 0)),
            pl.BlockSpec((BLOCK_ATOMS, D_FEAT), lambda i: (i, 0)),
            pl.BlockSpec((D_FEAT, D_OUT), lambda i: (0, 0)),
            pl.BlockSpec((1, D_OUT), lambda i: (0, 0)),
        ],
        out_specs=pl.BlockSpec((N_STRUCT, D_OUT), lambda i: (0, 0)),
        out_shape=jax.ShapeDtypeStruct((N_STRUCT, D_OUT), jnp.float32),
    )(ids3, x, W, b2)
    return out


# speedup vs baseline: 4.2136x; 1.3658x over previous
import jax
import jax.numpy as jnp
from jax.experimental import pallas as pl

N_ATOMS = 131072
D_FEAT = 512
D_OUT = 64
N_STRUCT = 1024

BLOCK_ATOMS = 2048
SEG_SEL = 128
SEG_STORE = SEG_SEL + 8


def _scatter_window(out_ref, yb16, ids, min_id, k):
    win_lo = min_id + k * SEG_SEL
    base = (jnp.minimum(win_lo, N_STRUCT - SEG_STORE) // 8) * 8
    rel = ids - base
    sel = (ids >= win_lo) & (ids < win_lo + SEG_SEL)
    rows = jax.lax.broadcasted_iota(jnp.int32, (SEG_STORE, BLOCK_ATOMS), 0)
    oh = ((rows == rel[None, :]) & sel[None, :]).astype(jnp.bfloat16)
    part = jnp.dot(oh, yb16, preferred_element_type=jnp.float32)
    out_ref[pl.ds(base, SEG_STORE), :] += part


def _fused_kernel(ids_ref, x_ref, w_ref, b_ref, out_ref):
    i = pl.program_id(0)

    @pl.when(i == 0)
    def _init():
        out_ref[...] = jnp.zeros_like(out_ref)

    xb16 = x_ref[...].astype(jnp.bfloat16)
    yb = jnp.dot(xb16, w_ref[...], preferred_element_type=jnp.float32)
    yb16 = (yb + b_ref[...]).astype(jnp.bfloat16)

    ids = ids_ref[0, 0, :]
    min_id = jnp.min(ids)
    max_id = jnp.max(ids)
    nwin = (max_id - min_id) // SEG_SEL + 1

    _scatter_window(out_ref, yb16, ids, min_id, 0)

    @pl.when(nwin > 1)
    def _rest():
        jax.lax.fori_loop(
            1, nwin,
            lambda k, c: (_scatter_window(out_ref, yb16, ids, min_id, k), c)[1],
            0)


@jax.jit
def kernel(x, structural_indices, W, b):
    nb = N_ATOMS // BLOCK_ATOMS
    ids3 = structural_indices.astype(jnp.int32).reshape(nb, 1, BLOCK_ATOMS)
    w16 = W.astype(jnp.bfloat16)
    b2 = b.reshape(1, D_OUT)
    out = pl.pallas_call(
        _fused_kernel,
        grid=(nb,),
        in_specs=[
            pl.BlockSpec((1, 1, BLOCK_ATOMS), lambda i: (i, 0, 0)),
            pl.BlockSpec((BLOCK_ATOMS, D_FEAT), lambda i: (i, 0)),
            pl.BlockSpec((D_FEAT, D_OUT), lambda i: (0, 0)),
            pl.BlockSpec((1, D_OUT), lambda i: (0, 0)),
        ],
        out_specs=pl.BlockSpec((N_STRUCT, D_OUT), lambda i: (0, 0)),
        out_shape=jax.ShapeDtypeStruct((N_STRUCT, D_OUT), jnp.float32),
    )(ids3, x, w16, b2)
    return out


# speedup vs baseline: 5.2996x; 1.2578x over previous
import jax
import jax.numpy as jnp
from jax.experimental import pallas as pl

N_ATOMS = 131072
D_FEAT = 512
D_OUT = 64
N_STRUCT = 1024

BLOCK_ATOMS = 4096
SEG_SEL = 64
SEG_STORE = SEG_SEL + 8


def _scatter_window(out_ref, yb16, ids, min_id, k):
    win_lo = min_id + k * SEG_SEL
    base = (jnp.minimum(win_lo, N_STRUCT - SEG_STORE) // 8) * 8
    rel = ids - base
    sel = (ids >= win_lo) & (ids < win_lo + SEG_SEL)
    rows = jax.lax.broadcasted_iota(jnp.int32, (SEG_STORE, BLOCK_ATOMS), 0)
    oh = ((rows == rel[None, :]) & sel[None, :]).astype(jnp.bfloat16)
    part = jnp.dot(oh, yb16, preferred_element_type=jnp.float32)
    out_ref[pl.ds(base, SEG_STORE), :] += part


def _fused_kernel(ids_ref, x_ref, w_ref, b_ref, out_ref):
    i = pl.program_id(0)

    @pl.when(i == 0)
    def _init():
        out_ref[...] = jnp.zeros_like(out_ref)

    xb16 = x_ref[...].astype(jnp.bfloat16)
    yb = jnp.dot(xb16, w_ref[...], preferred_element_type=jnp.float32)
    yb16 = (yb + b_ref[...]).astype(jnp.bfloat16)

    ids = ids_ref[0, 0, :]
    min_id = jnp.min(ids)
    max_id = jnp.max(ids)
    nwin = (max_id - min_id) // SEG_SEL + 1

    _scatter_window(out_ref, yb16, ids, min_id, 0)

    @pl.when(nwin > 1)
    def _rest():
        jax.lax.fori_loop(
            1, nwin,
            lambda k, c: (_scatter_window(out_ref, yb16, ids, min_id, k), c)[1],
            0)


@jax.jit
def kernel(x, structural_indices, W, b):
    nb = N_ATOMS // BLOCK_ATOMS
    ids3 = structural_indices.astype(jnp.int32).reshape(nb, 1, BLOCK_ATOMS)
    w16 = W.astype(jnp.bfloat16)
    b2 = b.reshape(1, D_OUT)
    out = pl.pallas_call(
        _fused_kernel,
        grid=(nb,),
        in_specs=[
            pl.BlockSpec((1, 1, BLOCK_ATOMS), lambda i: (i, 0, 0)),
            pl.BlockSpec((BLOCK_ATOMS, D_FEAT), lambda i: (i, 0)),
            pl.BlockSpec((D_FEAT, D_OUT), lambda i: (0, 0)),
            pl.BlockSpec((1, D_OUT), lambda i: (0, 0)),
        ],
        out_specs=pl.BlockSpec((N_STRUCT, D_OUT), lambda i: (0, 0)),
        out_shape=jax.ShapeDtypeStruct((N_STRUCT, D_OUT), jnp.float32),
    )(ids3, x, w16, b2)
    return out


# speedup vs baseline: 5.4446x; 1.0273x over previous
import jax
import jax.numpy as jnp
from jax.experimental import pallas as pl

N_ATOMS = 131072
D_FEAT = 512
D_OUT = 64
N_STRUCT = 1024

BLOCK_ATOMS = 8192
SEG_SEL = 64
SEG_STORE = SEG_SEL + 8


def _scatter_window(out_ref, yb16, ids, min_id, k):
    win_lo = min_id + k * SEG_SEL
    base = (jnp.minimum(win_lo, N_STRUCT - SEG_STORE) // 8) * 8
    rel = ids - base
    sel = (ids >= win_lo) & (ids < win_lo + SEG_SEL)
    rows = jax.lax.broadcasted_iota(jnp.int32, (SEG_STORE, BLOCK_ATOMS), 0)
    oh = ((rows == rel[None, :]) & sel[None, :]).astype(jnp.bfloat16)
    part = jnp.dot(oh, yb16, preferred_element_type=jnp.float32)
    out_ref[pl.ds(base, SEG_STORE), :] += part


def _fused_kernel(ids_ref, x_ref, w_ref, b_ref, out_ref):
    i = pl.program_id(0)

    @pl.when(i == 0)
    def _init():
        out_ref[...] = jnp.zeros_like(out_ref)

    xb16 = x_ref[...].astype(jnp.bfloat16)
    yb = jnp.dot(xb16, w_ref[...], preferred_element_type=jnp.float32)
    yb16 = (yb + b_ref[...]).astype(jnp.bfloat16)

    ids = ids_ref[0, 0, :]
    min_id = jnp.min(ids)
    max_id = jnp.max(ids)
    nwin = (max_id - min_id) // SEG_SEL + 1

    _scatter_window(out_ref, yb16, ids, min_id, 0)

    @pl.when(nwin > 1)
    def _rest():
        jax.lax.fori_loop(
            1, nwin,
            lambda k, c: (_scatter_window(out_ref, yb16, ids, min_id, k), c)[1],
            0)


@jax.jit
def kernel(x, structural_indices, W, b):
    nb = N_ATOMS // BLOCK_ATOMS
    ids3 = structural_indices.astype(jnp.int32).reshape(nb, 1, BLOCK_ATOMS)
    w16 = W.astype(jnp.bfloat16)
    b2 = b.reshape(1, D_OUT)
    out = pl.pallas_call(
        _fused_kernel,
        grid=(nb,),
        in_specs=[
            pl.BlockSpec((1, 1, BLOCK_ATOMS), lambda i: (i, 0, 0)),
            pl.BlockSpec((BLOCK_ATOMS, D_FEAT), lambda i: (i, 0)),
            pl.BlockSpec((D_FEAT, D_OUT), lambda i: (0, 0)),
            pl.BlockSpec((1, D_OUT), lambda i: (0, 0)),
        ],
        out_specs=pl.BlockSpec((N_STRUCT, D_OUT), lambda i: (0, 0)),
        out_shape=jax.ShapeDtypeStruct((N_STRUCT, D_OUT), jnp.float32),
    )(ids3, x, w16, b2)
    return out
